# Initial kernel scaffold; baseline (speedup 1.0000x reference)
import jax
import jax.numpy as jnp
from jax.experimental import pallas as pl


def kernel(pin_pos, netpin_start, flat_netpin, net_weights):
    raise NotImplementedError("write your pallas kernel here")


# speedup vs baseline: 80.3357x; 80.3357x over previous
import functools

import jax
import jax.numpy as jnp
from jax import lax
from jax.experimental import pallas as pl
from jax.experimental.pallas import tpu as pltpu
from jax.experimental.pallas import tpu_sc as plsc

NB = 512
BS = 1.0 / NB
NUM_NETS = 20000
PPN = 5
NC, NS, L = 2, 16, 16
NW = NC * NS
NETS_PAD = 20480
NETS_PER_W = NETS_PAD // NW
CHUNKS = NETS_PER_W // L
PINS_PER_W = NETS_PER_W * PPN
GDMA = 128
STR = 528
GRID = STR * STR
TSLICE = GRID // NS
OUT_FLAT = NC * 2 * GRID

_SCALE = float(NB) * float(NB) / 1.5625


def _sc_body(pin_hbm, ix_hbm, iy_hbm, w_hbm, z_hbm, part_hbm,
             gh_s, gv_s, ixb, iyb, pxb, pyb, wb, sidx, svh, svv, zb, sem):
    cid = lax.axis_index("c")
    sid = lax.axis_index("s")
    wid = sid * NC + cid
    pin_base = wid * PINS_PER_W
    net_base = wid * NETS_PER_W

    pltpu.sync_copy(ix_hbm.at[pl.ds(pin_base, PINS_PER_W)], ixb)
    pltpu.sync_copy(iy_hbm.at[pl.ds(pin_base, PINS_PER_W)], iyb)
    pltpu.sync_copy(w_hbm.at[pl.ds(net_base, NETS_PER_W)], wb)
    pltpu.sync_copy(z_hbm, zb)
    pltpu.sync_copy(zb, gh_s.at[pl.ds(sid * TSLICE, TSLICE)])
    pltpu.sync_copy(zb, gv_s.at[pl.ds(sid * TSLICE, TSLICE)])

    descs = []
    for j in range(PINS_PER_W // GDMA):
        s = pl.ds(j * GDMA, GDMA)
        descs.append(pltpu.async_copy(pin_hbm.at[ixb.at[s]], pxb.at[s], sem))
        descs.append(pltpu.async_copy(pin_hbm.at[iyb.at[s]], pyb.at[s], sem))
    for dsc in descs:
        dsc.wait()

    plsc.subcore_barrier()

    bsf = jnp.float32(BS)
    one = jnp.float32(1.0)

    def chunk(c, carry):
        pb = c * (L * PPN)
        xs = [pxb[pl.ds(pb + t * L, L)] for t in range(PPN)]
        ys = [pyb[pl.ds(pb + t * L, L)] for t in range(PPN)]
        xmin, xmax = xs[0], xs[0]
        ymin, ymax = ys[0], ys[0]
        for t in range(1, PPN):
            xmin = jnp.minimum(xmin, xs[t])
            xmax = jnp.maximum(xmax, xs[t])
            ymin = jnp.minimum(ymin, ys[t])
            ymax = jnp.maximum(ymax, ys[t])
        w = wb[pl.ds(c * L, L)]

        def atoms(lo, hi):
            sl = lo * jnp.float32(NB)
            kl = sl.astype(jnp.int32)
            fl = sl - kl.astype(jnp.float32)
            sh = hi * jnp.float32(NB)
            kh = sh.astype(jnp.int32)
            fh = sh - kh.astype(jnp.float32)
            ks = [kl, kl + 1, kh, kh + 1]
            vs = [(one - fl) * bsf, fl * bsf, (fh - one) * bsf, -(fh * bsf)]
            return ks, vs

        kx, vx = atoms(xmin, xmax)
        ky, vy = atoms(ymin, ymax)
        wh = w / (ymax - ymin + bsf)
        wv = w / (xmax - xmin + bsf)
        rowoff = [k * STR for k in kx]
        whx = [wh * v for v in vx]
        wvx = [wv * v for v in vx]
        for a in range(4):
            for b in range(4):
                p = a * 4 + b
                dst = pl.ds((p % 8) * L, L)
                sidx[p // 8, dst] = rowoff[a] + ky[b]
                svh[p // 8, dst] = whx[a] * vy[b]
                svv[p // 8, dst] = wvx[a] * vy[b]
        for j in range(2):
            pltpu.sync_copy(svh.at[j], gh_s.at[sidx.at[j]], add=True)
            pltpu.sync_copy(svv.at[j], gv_s.at[sidx.at[j]], add=True)
        return carry

    lax.fori_loop(0, CHUNKS, chunk, 0)

    plsc.subcore_barrier()

    obase = cid * (2 * GRID) + sid * TSLICE
    pltpu.sync_copy(gh_s.at[pl.ds(sid * TSLICE, TSLICE)], zb)
    pltpu.sync_copy(zb, part_hbm.at[pl.ds(obase, TSLICE)])
    pltpu.sync_copy(gv_s.at[pl.ds(sid * TSLICE, TSLICE)], zb)
    pltpu.sync_copy(zb, part_hbm.at[pl.ds(obase + GRID, TSLICE)])


_sc_call = pl.kernel(
    _sc_body,
    out_type=jax.ShapeDtypeStruct((OUT_FLAT,), jnp.float32),
    mesh=plsc.VectorSubcoreMesh(core_axis_name="c", subcore_axis_name="s"),
    scratch_types=[
        pltpu.VMEM_SHARED((GRID,), jnp.float32),
        pltpu.VMEM_SHARED((GRID,), jnp.float32),
        pltpu.VMEM((PINS_PER_W,), jnp.int32),
        pltpu.VMEM((PINS_PER_W,), jnp.int32),
        pltpu.VMEM((PINS_PER_W,), jnp.float32),
        pltpu.VMEM((PINS_PER_W,), jnp.float32),
        pltpu.VMEM((NETS_PER_W,), jnp.float32),
        pltpu.VMEM((2, GDMA), jnp.int32),
        pltpu.VMEM((2, GDMA), jnp.float32),
        pltpu.VMEM((2, GDMA), jnp.float32),
        pltpu.VMEM((TSLICE,), jnp.float32),
        pltpu.SemaphoreType.DMA,
    ],
)


def _tc_body(p_ref, o_ref):
    gh = p_ref[0, 0] + p_ref[1, 0]
    gv = p_ref[0, 1] + p_ref[1, 1]
    ii = lax.broadcasted_iota(jnp.int32, (NB, STR), 0)
    jj = lax.broadcasted_iota(jnp.int32, (NB, STR), 1)
    tri = (jj <= ii).astype(jnp.float32)
    dn_rows = (((1,), (0,)), ((), ()))
    dn_cols = (((1,), (1,)), ((), ()))
    mh = lax.dot_general(
        lax.dot_general(tri, gh, dn_rows, preferred_element_type=jnp.float32),
        tri, dn_cols, preferred_element_type=jnp.float32)
    mv = lax.dot_general(
        lax.dot_general(tri, gv, dn_rows, preferred_element_type=jnp.float32),
        tri, dn_cols, preferred_element_type=jnp.float32)
    o_ref[...] = jnp.maximum(jnp.abs(mh), jnp.abs(mv)) * jnp.float32(_SCALE)


_tc_call = pl.pallas_call(
    _tc_body,
    out_shape=jax.ShapeDtypeStruct((NB, NB), jnp.float32),
)


def kernel(pin_pos, netpin_start, flat_netpin, net_weights):
    num_pins = flat_netpin.shape[0]
    ix = jnp.pad(flat_netpin, (0, NETS_PAD * PPN - num_pins))
    ix = ix.reshape(NW, CHUNKS, L, PPN).transpose(0, 1, 3, 2).reshape(-1)
    iy = ix + num_pins
    w = jnp.pad(net_weights, (0, NETS_PAD - NUM_NETS))
    z = jnp.zeros((TSLICE,), jnp.float32)
    part = _sc_call(pin_pos, ix, iy, w, z)
    return _tc_call(part.reshape(NC, 2, STR, STR))


# speedup vs baseline: 89.5574x; 1.1148x over previous
import functools

import jax
import jax.numpy as jnp
from jax import lax
from jax.experimental import pallas as pl
from jax.experimental.pallas import tpu as pltpu
from jax.experimental.pallas import tpu_sc as plsc

NB = 512
BS = 1.0 / NB
NUM_NETS = 20000
PPN = 5
NC, NS, L = 2, 16, 16
NW = NC * NS
NETS_PAD = 20480
NETS_PER_W = NETS_PAD // NW
CHUNKS = NETS_PER_W // L
PINS_PER_W = NETS_PER_W * PPN
GDMA = 128
GRP = 4
STR = 528
GRID = STR * STR
TSLICE = GRID // NS
OUT_FLAT = NC * 2 * GRID

_SCALE = float(NB) * float(NB) / 1.5625


def _sc_body(pin_hbm, ix_hbm, iy_hbm, w_hbm, z_hbm, part_hbm,
             gh_s, gv_s, ixb, iyb, pxb, pyb, wb, sidx, svh, svv, zb, sem):
    cid = lax.axis_index("c")
    sid = lax.axis_index("s")
    wid = sid * NC + cid
    pin_base = wid * PINS_PER_W
    net_base = wid * NETS_PER_W

    pltpu.sync_copy(ix_hbm.at[pl.ds(pin_base, PINS_PER_W)], ixb)
    pltpu.sync_copy(iy_hbm.at[pl.ds(pin_base, PINS_PER_W)], iyb)

    descs = []
    for j in range(PINS_PER_W // GDMA):
        s = pl.ds(j * GDMA, GDMA)
        descs.append(pltpu.async_copy(pin_hbm.at[ixb.at[s]], pxb.at[s], sem))
        descs.append(pltpu.async_copy(pin_hbm.at[iyb.at[s]], pyb.at[s], sem))
    pltpu.sync_copy(w_hbm.at[pl.ds(net_base, NETS_PER_W)], wb)
    pltpu.sync_copy(z_hbm, zb)
    pltpu.sync_copy(zb, gh_s.at[pl.ds(sid * TSLICE, TSLICE)])
    pltpu.sync_copy(zb, gv_s.at[pl.ds(sid * TSLICE, TSLICE)])
    for dsc in descs:
        dsc.wait()

    plsc.subcore_barrier()

    bsf = jnp.float32(BS)
    one = jnp.float32(1.0)

    def group(g, carry):
        for cc in range(GRP):
            c = g * GRP + cc
            pb = c * (L * PPN)
            xs = [pxb[pl.ds(pb + t * L, L)] for t in range(PPN)]
            ys = [pyb[pl.ds(pb + t * L, L)] for t in range(PPN)]
            xmin, xmax = xs[0], xs[0]
            ymin, ymax = ys[0], ys[0]
            for t in range(1, PPN):
                xmin = jnp.minimum(xmin, xs[t])
                xmax = jnp.maximum(xmax, xs[t])
                ymin = jnp.minimum(ymin, ys[t])
                ymax = jnp.maximum(ymax, ys[t])
            w = wb[pl.ds(c * L, L)]

            def atoms(lo, hi):
                sl = lo * jnp.float32(NB)
                kl = sl.astype(jnp.int32)
                fl = sl - kl.astype(jnp.float32)
                sh = hi * jnp.float32(NB)
                kh = sh.astype(jnp.int32)
                fh = sh - kh.astype(jnp.float32)
                ks = [kl, kl + 1, kh, kh + 1]
                vs = [(one - fl) * bsf, fl * bsf, (fh - one) * bsf, -(fh * bsf)]
                return ks, vs

            kx, vx = atoms(xmin, xmax)
            ky, vy = atoms(ymin, ymax)
            wh = w / (ymax - ymin + bsf)
            wv = w / (xmax - xmin + bsf)
            rowoff = [k * STR for k in kx]
            whx = [wh * v for v in vx]
            wvx = [wv * v for v in vx]
            for a in range(4):
                for b in range(4):
                    p = a * 4 + b
                    row = cc * 2 + p // 8
                    dst = pl.ds((p % 8) * L, L)
                    sidx[row, dst] = rowoff[a] + ky[b]
                    svh[row, dst] = whx[a] * vy[b]
                    svv[row, dst] = wvx[a] * vy[b]
        descs = []
        for j in range(2 * GRP):
            descs.append(
                pltpu.async_copy(svh.at[j], gh_s.at[sidx.at[j]], sem, add=True))
            descs.append(
                pltpu.async_copy(svv.at[j], gv_s.at[sidx.at[j]], sem, add=True))
        for dsc in descs:
            dsc.wait()
        return carry

    lax.fori_loop(0, CHUNKS // GRP, group, 0)

    plsc.subcore_barrier()

    obase = cid * (2 * GRID) + sid * TSLICE
    pltpu.sync_copy(gh_s.at[pl.ds(sid * TSLICE, TSLICE)], zb)
    pltpu.sync_copy(zb, part_hbm.at[pl.ds(obase, TSLICE)])
    pltpu.sync_copy(gv_s.at[pl.ds(sid * TSLICE, TSLICE)], zb)
    pltpu.sync_copy(zb, part_hbm.at[pl.ds(obase + GRID, TSLICE)])


_sc_call = pl.kernel(
    _sc_body,
    out_type=jax.ShapeDtypeStruct((OUT_FLAT,), jnp.float32),
    mesh=plsc.VectorSubcoreMesh(core_axis_name="c", subcore_axis_name="s"),
    scratch_types=[
        pltpu.VMEM_SHARED((GRID,), jnp.float32),
        pltpu.VMEM_SHARED((GRID,), jnp.float32),
        pltpu.VMEM((PINS_PER_W,), jnp.int32),
        pltpu.VMEM((PINS_PER_W,), jnp.int32),
        pltpu.VMEM((PINS_PER_W,), jnp.float32),
        pltpu.VMEM((PINS_PER_W,), jnp.float32),
        pltpu.VMEM((NETS_PER_W,), jnp.float32),
        pltpu.VMEM((2 * GRP, GDMA), jnp.int32),
        pltpu.VMEM((2 * GRP, GDMA), jnp.float32),
        pltpu.VMEM((2 * GRP, GDMA), jnp.float32),
        pltpu.VMEM((TSLICE,), jnp.float32),
        pltpu.SemaphoreType.DMA,
    ],
)


def _tc_body(p_ref, o_ref):
    gh = p_ref[0, 0] + p_ref[1, 0]
    gv = p_ref[0, 1] + p_ref[1, 1]
    ii = lax.broadcasted_iota(jnp.int32, (NB, STR), 0)
    jj = lax.broadcasted_iota(jnp.int32, (NB, STR), 1)
    tri = (jj <= ii).astype(jnp.float32)
    dn_rows = (((1,), (0,)), ((), ()))
    dn_cols = (((1,), (1,)), ((), ()))
    mh = lax.dot_general(
        lax.dot_general(tri, gh, dn_rows, preferred_element_type=jnp.float32),
        tri, dn_cols, preferred_element_type=jnp.float32)
    mv = lax.dot_general(
        lax.dot_general(tri, gv, dn_rows, preferred_element_type=jnp.float32),
        tri, dn_cols, preferred_element_type=jnp.float32)
    o_ref[...] = jnp.maximum(jnp.abs(mh), jnp.abs(mv)) * jnp.float32(_SCALE)


_tc_call = pl.pallas_call(
    _tc_body,
    out_shape=jax.ShapeDtypeStruct((NB, NB), jnp.float32),
)


def kernel(pin_pos, netpin_start, flat_netpin, net_weights):
    num_pins = flat_netpin.shape[0]
    ix = jnp.pad(flat_netpin, (0, NETS_PAD * PPN - num_pins))
    ix = ix.reshape(NW, CHUNKS, L, PPN).transpose(0, 1, 3, 2).reshape(-1)
    iy = ix + num_pins
    w = jnp.pad(net_weights, (0, NETS_PAD - NUM_NETS))
    z = jnp.zeros((TSLICE,), jnp.float32)
    part = _sc_call(pin_pos, ix, iy, w, z)
    return _tc_call(part.reshape(NC, 2, STR, STR))
